# baseline (speedup 1.0000x reference)
import functools

import jax
import jax.numpy as jnp
from jax import lax
from jax.experimental import pallas as pl
from jax.experimental.pallas import tpu as pltpu
from jax.experimental.pallas import tpu_sc as plsc

_MOM = 0.95
_CAP_F = 1.25


def _pack2(a, b):
    col = lax.broadcasted_iota(jnp.int32, (a.shape[0], 2), 1)
    return jnp.where(col == 0, a, b)


def _router_body(x_ref, wg_ref, ema_ref, probs_ref, idx_ref, d0_ref, d1_ref,
                 s0_ref, s1_ref, wv_ref, ema_out_ref, *, T, E, CAP, R):
    xf = x_ref[...]
    wg = wg_ref[...]
    logits = jnp.dot(xf, wg, preferred_element_type=jnp.float32)
    m = jnp.max(logits, axis=1, keepdims=True)
    p = jnp.exp(logits - m)
    probs_full = p / jnp.sum(p, axis=1, keepdims=True)

    ei = lax.broadcasted_iota(jnp.int32, (T, E), 1)
    v1 = jnp.max(probs_full, axis=1, keepdims=True)
    i1 = jnp.min(jnp.where(probs_full == v1, ei, E), axis=1, keepdims=True)
    masked = jnp.where(ei == i1, jnp.float32(-1.0), probs_full)
    v2 = jnp.max(masked, axis=1, keepdims=True)
    i2 = jnp.min(jnp.where(masked == v2, ei, E), axis=1, keepdims=True)

    oh1 = (ei == i1).astype(jnp.float32)
    oh2 = (ei == i2).astype(jnp.float32)

    ri = lax.broadcasted_iota(jnp.int32, (R, R), 0)
    ci = lax.broadcasted_iota(jnp.int32, (R, R), 1)
    strict = (ci < ri).astype(jnp.float32)
    c1 = jnp.zeros((1, E), jnp.float32)
    c2 = jnp.zeros((1, E), jnp.float32)
    r1_parts = []
    r2_parts = []
    for b in range(T // R):
        m1 = oh1[b * R:(b + 1) * R]
        m2 = oh2[b * R:(b + 1) * R]
        rb1 = jnp.dot(strict, m1, preferred_element_type=jnp.float32) + c1
        rb2 = jnp.dot(strict, m2, preferred_element_type=jnp.float32) + c2
        r1_parts.append(jnp.sum(m1 * rb1, axis=1, keepdims=True))
        r2_parts.append(jnp.sum(m2 * rb2, axis=1, keepdims=True))
        c1 = c1 + jnp.sum(m1, axis=0, keepdims=True)
        c2 = c2 + jnp.sum(m2, axis=0, keepdims=True)

    rows_e = 2 * CAP
    trash = E * rows_e
    d1_parts, d2_parts, s1_parts, s2_parts, w1_parts, w2_parts = [], [], [], [], [], []
    for b in range(T // R):
        sl = slice(b * R, (b + 1) * R)
        r1i = r1_parts[b].astype(jnp.int32)
        r2i = r2_parts[b].astype(jnp.int32)
        ok1 = r1i < CAP
        ok2 = r2i < CAP
        row1 = i1[sl] * rows_e + r1i
        row2 = i2[sl] * rows_e + CAP + r2i
        d1_parts.append(jnp.where(ok1, row1, trash))
        d2_parts.append(jnp.where(ok2, row2, trash))
        s1_parts.append(jnp.where(ok1, row1, 0))
        s2_parts.append(jnp.where(ok2, row2, 0))
        w1_parts.append(jnp.where(ok1, v1[sl], 0.0))
        w2_parts.append(jnp.where(ok2, v2[sl], 0.0))

    for ref, parts in ((d0_ref, d1_parts), (d1_ref, d2_parts),
                       (s0_ref, s1_parts), (s1_ref, s2_parts)):
        col = jnp.concatenate(parts, axis=0)
        ref[...] = col.reshape(T // 128, 128)
    for b in range(T // R):
        wv_ref[b * R:(b + 1) * R, :] = _pack2(w1_parts[b], w2_parts[b])

    probs_ref[...] = _pack2(v1, v2)
    idx_ref[...] = _pack2(i1, i2)

    cnt = c1 + c2
    load = cnt / jnp.float32(2 * T)
    ema_out_ref[...] = _MOM * ema_ref[...] + (1.0 - _MOM) * load


def _dispatch_body(xf_hbm, d0_hbm, d1_hbm, xd_hbm, i0_v, i1_v, rows_v, sem,
                   *, CH, NC):
    wid = lax.axis_index("s") * NC + lax.axis_index("c")
    base = wid * CH
    pltpu.sync_copy(d0_hbm.at[pl.ds(base, CH)], i0_v)
    pltpu.sync_copy(d1_hbm.at[pl.ds(base, CH)], i1_v)
    pltpu.sync_copy(xf_hbm.at[pl.ds(base, CH)], rows_v)
    c0 = pltpu.async_copy(rows_v, xd_hbm.at[i0_v], sem)
    c1 = pltpu.async_copy(rows_v, xd_hbm.at[i1_v], sem)
    c0.wait()
    c1.wait()


def _gatherback_body(ty_hbm, s0_hbm, s1_hbm, g0_hbm, g1_hbm, i0_v, i1_v,
                     rows0_v, rows1_v, sem, *, CH, NC):
    wid = lax.axis_index("s") * NC + lax.axis_index("c")
    base = wid * CH
    pltpu.sync_copy(s0_hbm.at[pl.ds(base, CH)], i0_v)
    pltpu.sync_copy(s1_hbm.at[pl.ds(base, CH)], i1_v)
    c0 = pltpu.async_copy(ty_hbm.at[i0_v], rows0_v, sem)
    c1 = pltpu.async_copy(ty_hbm.at[i1_v], rows1_v, sem)
    c0.wait()
    c1.wait()
    pltpu.sync_copy(rows0_v, g0_hbm.at[pl.ds(base, CH)])
    pltpu.sync_copy(rows1_v, g1_hbm.at[pl.ds(base, CH)])


def _mlp_body(xd_ref, w1_ref, b1_ref, w2_ref, b2_ref, ty_ref):
    xb = xd_ref[0].astype(jnp.bfloat16)
    h = jnp.dot(xb, w1_ref[0].astype(jnp.bfloat16),
                preferred_element_type=jnp.float32) + b1_ref[0]
    h = 0.5 * h * (1.0 + lax.erf(h * jnp.float32(0.7071067811865476)))
    y = jnp.dot(h.astype(jnp.bfloat16), w2_ref[0].astype(jnp.bfloat16),
                preferred_element_type=jnp.float32) + b2_ref[0]
    ty_ref[0] = y


def _combine_body(g0_ref, g1_ref, wv_ref, out_ref):
    w0 = wv_ref[:, 0:1]
    w1 = wv_ref[:, 1:2]
    a = jnp.where(w0 > 0.0, w0 * g0_ref[...], 0.0)
    b = jnp.where(w1 > 0.0, w1 * g1_ref[...], 0.0)
    out_ref[...] = a + b


def kernel(x, Wg, W1, b1, W2, b2, ema_load):
    B, S, D = x.shape
    T = B * S
    E = Wg.shape[1]
    H = W1.shape[2]
    K = 2
    CAP = max(1, int(T * _CAP_F / E))
    ROWS_E = 2 * CAP
    NROWS = E * ROWS_E
    NPAD = NROWS + 8
    R = 256 if T % 256 == 0 else T
    xf = x.reshape(T, D)

    router = pl.pallas_call(
        functools.partial(_router_body, T=T, E=E, CAP=CAP, R=R),
        out_shape=[
            jax.ShapeDtypeStruct((T, 2), jnp.float32),
            jax.ShapeDtypeStruct((T, 2), jnp.int32),
            jax.ShapeDtypeStruct((T // 128, 128), jnp.int32),
            jax.ShapeDtypeStruct((T // 128, 128), jnp.int32),
            jax.ShapeDtypeStruct((T // 128, 128), jnp.int32),
            jax.ShapeDtypeStruct((T // 128, 128), jnp.int32),
            jax.ShapeDtypeStruct((T, 2), jnp.float32),
            jax.ShapeDtypeStruct((1, E), jnp.float32),
        ],
    )
    (probs2, idx2, dst0, dst1, src0, src1, wv2,
     ema2) = router(xf, Wg, ema_load.reshape(1, E))

    info = plsc.get_sparse_core_info()
    NC, NS = info.num_cores, info.num_subcores
    NW = NC * NS
    CH = T // NW
    mesh = plsc.VectorSubcoreMesh(core_axis_name="c", subcore_axis_name="s")

    dispatch = pl.kernel(
        functools.partial(_dispatch_body, CH=CH, NC=NC),
        out_type=jax.ShapeDtypeStruct((NPAD, D), jnp.float32),
        mesh=mesh,
        scratch_types=[
            pltpu.VMEM((CH,), jnp.int32),
            pltpu.VMEM((CH,), jnp.int32),
            pltpu.VMEM((CH, D), jnp.float32),
            pltpu.SemaphoreType.DMA,
        ],
    )
    xd = dispatch(xf, dst0.reshape(T), dst1.reshape(T))

    mlp = pl.pallas_call(
        _mlp_body,
        grid=(E,),
        in_specs=[
            pl.BlockSpec((1, ROWS_E, D), lambda e: (e, 0, 0)),
            pl.BlockSpec((1, D, H), lambda e: (e, 0, 0)),
            pl.BlockSpec((1, 1, H), lambda e: (e, 0, 0)),
            pl.BlockSpec((1, H, D), lambda e: (e, 0, 0)),
            pl.BlockSpec((1, 1, D), lambda e: (e, 0, 0)),
        ],
        out_specs=pl.BlockSpec((1, ROWS_E, D), lambda e: (e, 0, 0)),
        out_shape=jax.ShapeDtypeStruct((E, ROWS_E, D), jnp.float32),
    )
    ty = mlp(xd[:NROWS].reshape(E, ROWS_E, D), W1, b1.reshape(E, 1, H),
             W2, b2.reshape(E, 1, D))

    gatherback = pl.kernel(
        functools.partial(_gatherback_body, CH=CH, NC=NC),
        out_type=[
            jax.ShapeDtypeStruct((T, D), jnp.float32),
            jax.ShapeDtypeStruct((T, D), jnp.float32),
        ],
        mesh=mesh,
        scratch_types=[
            pltpu.VMEM((CH,), jnp.int32),
            pltpu.VMEM((CH,), jnp.int32),
            pltpu.VMEM((CH, D), jnp.float32),
            pltpu.VMEM((CH, D), jnp.float32),
            pltpu.SemaphoreType.DMA,
        ],
    )
    g0, g1 = gatherback(ty.reshape(NROWS, D), src0.reshape(T), src1.reshape(T))

    RB = 256 if T % 256 == 0 else T
    combine = pl.pallas_call(
        _combine_body,
        grid=(T // RB,),
        in_specs=[
            pl.BlockSpec((RB, D), lambda i: (i, 0)),
            pl.BlockSpec((RB, D), lambda i: (i, 0)),
            pl.BlockSpec((RB, 2), lambda i: (i, 0)),
        ],
        out_specs=pl.BlockSpec((RB, D), lambda i: (i, 0)),
        out_shape=jax.ShapeDtypeStruct((T, D), jnp.float32),
    )
    out = combine(g0, g1, wv2)

    return (out.reshape(B, S, D), probs2.reshape(B, S, K),
            idx2.reshape(B, S, K), ema2.reshape(E))


# speedup vs baseline: 1.0881x; 1.0881x over previous
import functools

import jax
import jax.numpy as jnp
from jax import lax
from jax.experimental import pallas as pl
from jax.experimental.pallas import tpu as pltpu
from jax.experimental.pallas import tpu_sc as plsc

_MOM = 0.95
_CAP_F = 1.25


def _pack2(a, b):
    col = lax.broadcasted_iota(jnp.int32, (a.shape[0], 2), 1)
    return jnp.where(col == 0, a, b)


def _router_body(x_ref, wg_ref, ema_ref, probs_ref, idx_ref, d0_ref, d1_ref,
                 s0_ref, s1_ref, wv_ref, ema_out_ref, *, T, E, CAP, R):
    xf = x_ref[...]
    wg = wg_ref[...]
    logits = jnp.dot(xf, wg, preferred_element_type=jnp.float32)
    m = jnp.max(logits, axis=1, keepdims=True)
    p = jnp.exp(logits - m)
    probs_full = p / jnp.sum(p, axis=1, keepdims=True)

    ei = lax.broadcasted_iota(jnp.int32, (T, E), 1)
    v1 = jnp.max(probs_full, axis=1, keepdims=True)
    i1 = jnp.min(jnp.where(probs_full == v1, ei, E), axis=1, keepdims=True)
    masked = jnp.where(ei == i1, jnp.float32(-1.0), probs_full)
    v2 = jnp.max(masked, axis=1, keepdims=True)
    i2 = jnp.min(jnp.where(masked == v2, ei, E), axis=1, keepdims=True)

    oh1 = (ei == i1).astype(jnp.float32)
    oh2 = (ei == i2).astype(jnp.float32)

    ri = lax.broadcasted_iota(jnp.int32, (R, R), 0)
    ci = lax.broadcasted_iota(jnp.int32, (R, R), 1)
    strict = (ci < ri).astype(jnp.float32)
    c1 = jnp.zeros((1, E), jnp.float32)
    c2 = jnp.zeros((1, E), jnp.float32)
    r1_parts = []
    r2_parts = []
    for b in range(T // R):
        m1 = oh1[b * R:(b + 1) * R]
        m2 = oh2[b * R:(b + 1) * R]
        rb1 = jnp.dot(strict, m1, preferred_element_type=jnp.float32) + c1
        rb2 = jnp.dot(strict, m2, preferred_element_type=jnp.float32) + c2
        r1_parts.append(jnp.sum(m1 * rb1, axis=1, keepdims=True))
        r2_parts.append(jnp.sum(m2 * rb2, axis=1, keepdims=True))
        c1 = c1 + jnp.sum(m1, axis=0, keepdims=True)
        c2 = c2 + jnp.sum(m2, axis=0, keepdims=True)

    rows_e = 2 * CAP + 8
    d1_parts, d2_parts, s1_parts, s2_parts, w1_parts, w2_parts = [], [], [], [], [], []
    for b in range(T // R):
        sl = slice(b * R, (b + 1) * R)
        r1i = r1_parts[b].astype(jnp.int32)
        r2i = r2_parts[b].astype(jnp.int32)
        ok1 = r1i < CAP
        ok2 = r2i < CAP
        row1 = i1[sl] * rows_e + r1i
        row2 = i2[sl] * rows_e + CAP + r2i
        d1_parts.append(jnp.where(ok1, row1, i1[sl] * rows_e + 2 * CAP))
        d2_parts.append(jnp.where(ok2, row2, i2[sl] * rows_e + 2 * CAP))
        s1_parts.append(jnp.where(ok1, row1, 0))
        s2_parts.append(jnp.where(ok2, row2, 0))
        w1_parts.append(jnp.where(ok1, v1[sl], 0.0))
        w2_parts.append(jnp.where(ok2, v2[sl], 0.0))

    for ref, parts in ((d0_ref, d1_parts), (d1_ref, d2_parts),
                       (s0_ref, s1_parts), (s1_ref, s2_parts)):
        col = jnp.concatenate(parts, axis=0)
        ref[...] = col.reshape(T // 128, 128)
    for b in range(T // R):
        wv_ref[b * R:(b + 1) * R, :] = _pack2(w1_parts[b], w2_parts[b])

    probs_ref[...] = _pack2(v1, v2)
    idx_ref[...] = _pack2(i1, i2)

    cnt = c1 + c2
    load = cnt / jnp.float32(2 * T)
    ema_out_ref[...] = _MOM * ema_ref[...] + (1.0 - _MOM) * load


def _dispatch_body(xf_hbm, d0_hbm, d1_hbm, xd_hbm, i0_v, i1_v, rows_v, sem,
                   *, CH, NC):
    wid = lax.axis_index("s") * NC + lax.axis_index("c")
    base = wid * CH
    row = base // 128
    off = base - row * 128
    pltpu.sync_copy(d0_hbm.at[row, pl.ds(off, CH)], i0_v)
    pltpu.sync_copy(d1_hbm.at[row, pl.ds(off, CH)], i1_v)
    pltpu.sync_copy(xf_hbm.at[pl.ds(base, CH)], rows_v)
    c0 = pltpu.async_copy(rows_v, xd_hbm.at[i0_v], sem)
    c1 = pltpu.async_copy(rows_v, xd_hbm.at[i1_v], sem)
    c0.wait()
    c1.wait()


def _gatherback_body(ty_hbm, s0_hbm, s1_hbm, g0_hbm, g1_hbm, i0_v, i1_v,
                     rows0_v, rows1_v, sem, *, CH, NC):
    wid = lax.axis_index("s") * NC + lax.axis_index("c")
    base = wid * CH
    row = base // 128
    off = base - row * 128
    pltpu.sync_copy(s0_hbm.at[row, pl.ds(off, CH)], i0_v)
    pltpu.sync_copy(s1_hbm.at[row, pl.ds(off, CH)], i1_v)
    c0 = pltpu.async_copy(ty_hbm.at[i0_v], rows0_v, sem)
    c1 = pltpu.async_copy(ty_hbm.at[i1_v], rows1_v, sem)
    c0.wait()
    c1.wait()
    pltpu.sync_copy(rows0_v, g0_hbm.at[pl.ds(base, CH)])
    pltpu.sync_copy(rows1_v, g1_hbm.at[pl.ds(base, CH)])


def _mlp_body(xd_ref, w1_ref, b1_ref, w2_ref, b2_ref, ty_ref):
    xb = xd_ref[0].astype(jnp.bfloat16)
    h = jnp.dot(xb, w1_ref[0].astype(jnp.bfloat16),
                preferred_element_type=jnp.float32) + b1_ref[0]
    h = 0.5 * h * (1.0 + lax.erf(h * jnp.float32(0.7071067811865476)))
    y = jnp.dot(h.astype(jnp.bfloat16), w2_ref[0].astype(jnp.bfloat16),
                preferred_element_type=jnp.float32) + b2_ref[0]
    ty_ref[0] = y


def _combine_body(g0_ref, g1_ref, wv_ref, out_ref):
    w0 = wv_ref[:, 0:1]
    w1 = wv_ref[:, 1:2]
    a = jnp.where(w0 > 0.0, w0 * g0_ref[...], 0.0)
    b = jnp.where(w1 > 0.0, w1 * g1_ref[...], 0.0)
    out_ref[...] = a + b


def kernel(x, Wg, W1, b1, W2, b2, ema_load):
    B, S, D = x.shape
    T = B * S
    E = Wg.shape[1]
    H = W1.shape[2]
    K = 2
    CAP = max(1, int(T * _CAP_F / E))
    ROWS_E = 2 * CAP + 8
    NPAD = E * ROWS_E
    R = 256 if T % 256 == 0 else T
    xf = x.reshape(T, D)

    router = pl.pallas_call(
        functools.partial(_router_body, T=T, E=E, CAP=CAP, R=R),
        out_shape=[
            jax.ShapeDtypeStruct((T, 2), jnp.float32),
            jax.ShapeDtypeStruct((T, 2), jnp.int32),
            jax.ShapeDtypeStruct((T // 128, 128), jnp.int32),
            jax.ShapeDtypeStruct((T // 128, 128), jnp.int32),
            jax.ShapeDtypeStruct((T // 128, 128), jnp.int32),
            jax.ShapeDtypeStruct((T // 128, 128), jnp.int32),
            jax.ShapeDtypeStruct((T, 2), jnp.float32),
            jax.ShapeDtypeStruct((1, E), jnp.float32),
        ],
    )
    (probs2, idx2, dst0, dst1, src0, src1, wv2,
     ema2) = router(xf, Wg, ema_load.reshape(1, E))

    info = plsc.get_sparse_core_info()
    NC, NS = info.num_cores, info.num_subcores
    NW = NC * NS
    CH = T // NW
    mesh = plsc.VectorSubcoreMesh(core_axis_name="c", subcore_axis_name="s")

    dispatch = pl.kernel(
        functools.partial(_dispatch_body, CH=CH, NC=NC),
        out_type=jax.ShapeDtypeStruct((NPAD, D), jnp.float32),
        mesh=mesh,
        scratch_types=[
            pltpu.VMEM((CH,), jnp.int32),
            pltpu.VMEM((CH,), jnp.int32),
            pltpu.VMEM((CH, D), jnp.float32),
            pltpu.SemaphoreType.DMA,
        ],
    )
    xd = dispatch(xf, dst0, dst1)

    mlp = pl.pallas_call(
        _mlp_body,
        grid=(E,),
        in_specs=[
            pl.BlockSpec((1, ROWS_E, D), lambda e: (e, 0, 0)),
            pl.BlockSpec((1, D, H), lambda e: (e, 0, 0)),
            pl.BlockSpec((1, 1, H), lambda e: (e, 0, 0)),
            pl.BlockSpec((1, H, D), lambda e: (e, 0, 0)),
            pl.BlockSpec((1, 1, D), lambda e: (e, 0, 0)),
        ],
        out_specs=pl.BlockSpec((1, ROWS_E, D), lambda e: (e, 0, 0)),
        out_shape=jax.ShapeDtypeStruct((E, ROWS_E, D), jnp.float32),
    )
    ty = mlp(xd.reshape(E, ROWS_E, D), W1, b1.reshape(E, 1, H),
             W2, b2.reshape(E, 1, D))

    gatherback = pl.kernel(
        functools.partial(_gatherback_body, CH=CH, NC=NC),
        out_type=[
            jax.ShapeDtypeStruct((T, D), jnp.float32),
            jax.ShapeDtypeStruct((T, D), jnp.float32),
        ],
        mesh=mesh,
        scratch_types=[
            pltpu.VMEM((CH,), jnp.int32),
            pltpu.VMEM((CH,), jnp.int32),
            pltpu.VMEM((CH, D), jnp.float32),
            pltpu.VMEM((CH, D), jnp.float32),
            pltpu.SemaphoreType.DMA,
        ],
    )
    g0, g1 = gatherback(ty.reshape(NPAD, D), src0, src1)

    RB = 256 if T % 256 == 0 else T
    combine = pl.pallas_call(
        _combine_body,
        grid=(T // RB,),
        in_specs=[
            pl.BlockSpec((RB, D), lambda i: (i, 0)),
            pl.BlockSpec((RB, D), lambda i: (i, 0)),
            pl.BlockSpec((RB, 2), lambda i: (i, 0)),
        ],
        out_specs=pl.BlockSpec((RB, D), lambda i: (i, 0)),
        out_shape=jax.ShapeDtypeStruct((T, D), jnp.float32),
    )
    out = combine(g0, g1, wv2)

    return (out.reshape(B, S, D), probs2.reshape(B, S, K),
            idx2.reshape(B, S, K), ema2.reshape(E))


# speedup vs baseline: 1.0954x; 1.0067x over previous
import functools

import jax
import jax.numpy as jnp
from jax import lax
from jax.experimental import pallas as pl
from jax.experimental.pallas import tpu as pltpu
from jax.experimental.pallas import tpu_sc as plsc

_MOM = 0.95
_CAP_F = 1.25


def _pack2(a, b):
    col = lax.broadcasted_iota(jnp.int32, (a.shape[0], 2), 1)
    return jnp.where(col == 0, a, b)


def _router_body(x_ref, wg_ref, ema_ref, probs_ref, idx_ref, d0_ref, d1_ref,
                 s0_ref, s1_ref, wv_ref, ema_out_ref, *, T, E, CAP, R):
    xf = x_ref[...]
    wg = wg_ref[...]
    logits = jnp.dot(xf, wg, preferred_element_type=jnp.float32)
    m = jnp.max(logits, axis=1, keepdims=True)
    p = jnp.exp(logits - m)
    probs_full = p / jnp.sum(p, axis=1, keepdims=True)

    ei = lax.broadcasted_iota(jnp.int32, (T, E), 1)
    v1 = jnp.max(probs_full, axis=1, keepdims=True)
    i1 = jnp.min(jnp.where(probs_full == v1, ei, E), axis=1, keepdims=True)
    masked = jnp.where(ei == i1, jnp.float32(-1.0), probs_full)
    v2 = jnp.max(masked, axis=1, keepdims=True)
    i2 = jnp.min(jnp.where(masked == v2, ei, E), axis=1, keepdims=True)

    oh1 = (ei == i1).astype(jnp.float32)
    oh2 = (ei == i2).astype(jnp.float32)

    ri = lax.broadcasted_iota(jnp.int32, (R, R), 0)
    ci = lax.broadcasted_iota(jnp.int32, (R, R), 1)
    strict = (ci < ri).astype(jnp.float32)
    c1 = jnp.zeros((1, E), jnp.float32)
    c2 = jnp.zeros((1, E), jnp.float32)
    r1_parts = []
    r2_parts = []
    for b in range(T // R):
        m1 = oh1[b * R:(b + 1) * R]
        m2 = oh2[b * R:(b + 1) * R]
        rb1 = jnp.dot(strict, m1, preferred_element_type=jnp.float32) + c1
        rb2 = jnp.dot(strict, m2, preferred_element_type=jnp.float32) + c2
        r1_parts.append(jnp.sum(m1 * rb1, axis=1, keepdims=True))
        r2_parts.append(jnp.sum(m2 * rb2, axis=1, keepdims=True))
        c1 = c1 + jnp.sum(m1, axis=0, keepdims=True)
        c2 = c2 + jnp.sum(m2, axis=0, keepdims=True)

    rows_e = 2 * CAP + 8
    d1_parts, d2_parts, s1_parts, s2_parts, w1_parts, w2_parts = [], [], [], [], [], []
    for b in range(T // R):
        sl = slice(b * R, (b + 1) * R)
        r1i = r1_parts[b].astype(jnp.int32)
        r2i = r2_parts[b].astype(jnp.int32)
        ok1 = r1i < CAP
        ok2 = r2i < CAP
        row1 = i1[sl] * rows_e + r1i
        row2 = i2[sl] * rows_e + CAP + r2i
        d1_parts.append(jnp.where(ok1, row1, i1[sl] * rows_e + 2 * CAP))
        d2_parts.append(jnp.where(ok2, row2, i2[sl] * rows_e + 2 * CAP))
        s1_parts.append(jnp.where(ok1, row1, 0))
        s2_parts.append(jnp.where(ok2, row2, 0))
        w1_parts.append(jnp.where(ok1, v1[sl], 0.0))
        w2_parts.append(jnp.where(ok2, v2[sl], 0.0))

    for ref, parts in ((d0_ref, d1_parts), (d1_ref, d2_parts),
                       (s0_ref, s1_parts), (s1_ref, s2_parts)):
        col = jnp.concatenate(parts, axis=0)
        ref[...] = col.reshape(T // 128, 128)
    for b in range(T // R):
        wv_ref[b * R:(b + 1) * R, :] = _pack2(w1_parts[b], w2_parts[b])

    probs_ref[...] = _pack2(v1, v2)
    idx_ref[...] = _pack2(i1, i2)

    cnt = c1 + c2
    load = cnt / jnp.float32(2 * T)
    ema_out_ref[...] = _MOM * ema_ref[...] + (1.0 - _MOM) * load


def _dispatch_body(xf_hbm, d0_hbm, d1_hbm, xd_hbm, i0_v, i1_v, rows_v, sem,
                   *, CH, NC):
    wid = lax.axis_index("s") * NC + lax.axis_index("c")
    base = wid * CH
    row = base // 128
    off = base - row * 128
    l0 = pltpu.async_copy(d0_hbm.at[row, pl.ds(off, CH)], i0_v, sem)
    l1 = pltpu.async_copy(d1_hbm.at[row, pl.ds(off, CH)], i1_v, sem)
    l2 = pltpu.async_copy(xf_hbm.at[pl.ds(base, CH)], rows_v, sem)
    l0.wait()
    l1.wait()
    l2.wait()
    c0 = pltpu.async_copy(rows_v, xd_hbm.at[i0_v], sem)
    c1 = pltpu.async_copy(rows_v, xd_hbm.at[i1_v], sem)
    c0.wait()
    c1.wait()


def _gatherback_body(ty_hbm, s0_hbm, s1_hbm, g0_hbm, g1_hbm, i0_v, i1_v,
                     rows0_v, rows1_v, sem, *, CH, NC):
    wid = lax.axis_index("s") * NC + lax.axis_index("c")
    base = wid * CH
    row = base // 128
    off = base - row * 128
    l0 = pltpu.async_copy(s0_hbm.at[row, pl.ds(off, CH)], i0_v, sem)
    l1 = pltpu.async_copy(s1_hbm.at[row, pl.ds(off, CH)], i1_v, sem)
    l0.wait()
    l1.wait()
    c0 = pltpu.async_copy(ty_hbm.at[i0_v], rows0_v, sem)
    c1 = pltpu.async_copy(ty_hbm.at[i1_v], rows1_v, sem)
    c0.wait()
    c1.wait()
    w0 = pltpu.async_copy(rows0_v, g0_hbm.at[pl.ds(base, CH)], sem)
    w1 = pltpu.async_copy(rows1_v, g1_hbm.at[pl.ds(base, CH)], sem)
    w0.wait()
    w1.wait()


def _mlp_body(xd_ref, w1_ref, b1_ref, w2_ref, b2_ref, ty_ref):
    xb = xd_ref[0].astype(jnp.bfloat16)
    h = jnp.dot(xb, w1_ref[0].astype(jnp.bfloat16),
                preferred_element_type=jnp.float32) + b1_ref[0]
    h = 0.5 * h * (1.0 + lax.erf(h * jnp.float32(0.7071067811865476)))
    y = jnp.dot(h.astype(jnp.bfloat16), w2_ref[0].astype(jnp.bfloat16),
                preferred_element_type=jnp.float32) + b2_ref[0]
    ty_ref[0] = y


def _combine_body(g0_ref, g1_ref, wv_ref, out_ref):
    w0 = wv_ref[:, 0:1]
    w1 = wv_ref[:, 1:2]
    a = jnp.where(w0 > 0.0, w0 * g0_ref[...], 0.0)
    b = jnp.where(w1 > 0.0, w1 * g1_ref[...], 0.0)
    out_ref[...] = a + b


def kernel(x, Wg, W1, b1, W2, b2, ema_load):
    B, S, D = x.shape
    T = B * S
    E = Wg.shape[1]
    H = W1.shape[2]
    K = 2
    CAP = max(1, int(T * _CAP_F / E))
    ROWS_E = 2 * CAP + 8
    NPAD = E * ROWS_E
    R = 256 if T % 256 == 0 else T
    xf = x.reshape(T, D)

    router = pl.pallas_call(
        functools.partial(_router_body, T=T, E=E, CAP=CAP, R=R),
        out_shape=[
            jax.ShapeDtypeStruct((T, 2), jnp.float32),
            jax.ShapeDtypeStruct((T, 2), jnp.int32),
            jax.ShapeDtypeStruct((T // 128, 128), jnp.int32),
            jax.ShapeDtypeStruct((T // 128, 128), jnp.int32),
            jax.ShapeDtypeStruct((T // 128, 128), jnp.int32),
            jax.ShapeDtypeStruct((T // 128, 128), jnp.int32),
            jax.ShapeDtypeStruct((T, 2), jnp.float32),
            jax.ShapeDtypeStruct((1, E), jnp.float32),
        ],
    )
    (probs2, idx2, dst0, dst1, src0, src1, wv2,
     ema2) = router(xf, Wg, ema_load.reshape(1, E))

    info = plsc.get_sparse_core_info()
    NC, NS = info.num_cores, info.num_subcores
    NW = NC * NS
    CH = T // NW
    mesh = plsc.VectorSubcoreMesh(core_axis_name="c", subcore_axis_name="s")

    dispatch = pl.kernel(
        functools.partial(_dispatch_body, CH=CH, NC=NC),
        out_type=jax.ShapeDtypeStruct((NPAD, D), jnp.float32),
        mesh=mesh,
        scratch_types=[
            pltpu.VMEM((CH,), jnp.int32),
            pltpu.VMEM((CH,), jnp.int32),
            pltpu.VMEM((CH, D), jnp.float32),
            pltpu.SemaphoreType.DMA,
        ],
    )
    xd = dispatch(xf, dst0, dst1)

    mlp = pl.pallas_call(
        _mlp_body,
        grid=(E,),
        in_specs=[
            pl.BlockSpec((1, ROWS_E, D), lambda e: (e, 0, 0)),
            pl.BlockSpec((1, D, H), lambda e: (e, 0, 0)),
            pl.BlockSpec((1, 1, H), lambda e: (e, 0, 0)),
            pl.BlockSpec((1, H, D), lambda e: (e, 0, 0)),
            pl.BlockSpec((1, 1, D), lambda e: (e, 0, 0)),
        ],
        out_specs=pl.BlockSpec((1, ROWS_E, D), lambda e: (e, 0, 0)),
        out_shape=jax.ShapeDtypeStruct((E, ROWS_E, D), jnp.float32),
    )
    ty = mlp(xd.reshape(E, ROWS_E, D), W1, b1.reshape(E, 1, H),
             W2, b2.reshape(E, 1, D))

    gatherback = pl.kernel(
        functools.partial(_gatherback_body, CH=CH, NC=NC),
        out_type=[
            jax.ShapeDtypeStruct((T, D), jnp.float32),
            jax.ShapeDtypeStruct((T, D), jnp.float32),
        ],
        mesh=mesh,
        scratch_types=[
            pltpu.VMEM((CH,), jnp.int32),
            pltpu.VMEM((CH,), jnp.int32),
            pltpu.VMEM((CH, D), jnp.float32),
            pltpu.VMEM((CH, D), jnp.float32),
            pltpu.SemaphoreType.DMA,
        ],
    )
    g0, g1 = gatherback(ty.reshape(NPAD, D), src0, src1)

    RB = 256 if T % 256 == 0 else T
    combine = pl.pallas_call(
        _combine_body,
        grid=(T // RB,),
        in_specs=[
            pl.BlockSpec((RB, D), lambda i: (i, 0)),
            pl.BlockSpec((RB, D), lambda i: (i, 0)),
            pl.BlockSpec((RB, 2), lambda i: (i, 0)),
        ],
        out_specs=pl.BlockSpec((RB, D), lambda i: (i, 0)),
        out_shape=jax.ShapeDtypeStruct((T, D), jnp.float32),
    )
    out = combine(g0, g1, wv2)

    return (out.reshape(B, S, D), probs2.reshape(B, S, K),
            idx2.reshape(B, S, K), ema2.reshape(E))


# speedup vs baseline: 1.0995x; 1.0037x over previous
import functools

import jax
import jax.numpy as jnp
from jax import lax
from jax.experimental import pallas as pl
from jax.experimental.pallas import tpu as pltpu
from jax.experimental.pallas import tpu_sc as plsc

_MOM = 0.95
_CAP_F = 1.25


def _pack2(a, b):
    col = lax.broadcasted_iota(jnp.int32, (a.shape[0], 2), 1)
    return jnp.where(col == 0, a, b)


def _router_body(x_ref, wg_ref, ema_ref, probs_ref, idx_ref, d0_ref, d1_ref,
                 s0_ref, s1_ref, wv_ref, ema_out_ref, c1_ref, c2_ref,
                 *, T, E, CAP, R, NB):
    b = pl.program_id(0)

    @pl.when(b == 0)
    def _init():
        c1_ref[...] = jnp.zeros((1, E), jnp.float32)
        c2_ref[...] = jnp.zeros((1, E), jnp.float32)

    xb = x_ref[...]
    wg = wg_ref[...]
    logits = jnp.dot(xb, wg, preferred_element_type=jnp.float32)
    m = jnp.max(logits, axis=1, keepdims=True)
    p = jnp.exp(logits - m)
    probs_full = p / jnp.sum(p, axis=1, keepdims=True)

    ei = lax.broadcasted_iota(jnp.int32, (R, E), 1)
    v1 = jnp.max(probs_full, axis=1, keepdims=True)
    i1 = jnp.min(jnp.where(probs_full == v1, ei, E), axis=1, keepdims=True)
    masked = jnp.where(ei == i1, jnp.float32(-1.0), probs_full)
    v2 = jnp.max(masked, axis=1, keepdims=True)
    i2 = jnp.min(jnp.where(masked == v2, ei, E), axis=1, keepdims=True)

    oh1 = (ei == i1).astype(jnp.float32)
    oh2 = (ei == i2).astype(jnp.float32)

    ri = lax.broadcasted_iota(jnp.int32, (R, R), 0)
    ci = lax.broadcasted_iota(jnp.int32, (R, R), 1)
    strict = (ci < ri).astype(jnp.float32)
    c1 = c1_ref[...]
    c2 = c2_ref[...]
    rb1 = jnp.dot(strict, oh1, preferred_element_type=jnp.float32) + c1
    rb2 = jnp.dot(strict, oh2, preferred_element_type=jnp.float32) + c2
    r1i = jnp.sum(oh1 * rb1, axis=1, keepdims=True).astype(jnp.int32)
    r2i = jnp.sum(oh2 * rb2, axis=1, keepdims=True).astype(jnp.int32)
    c1 = c1 + jnp.sum(oh1, axis=0, keepdims=True)
    c2 = c2 + jnp.sum(oh2, axis=0, keepdims=True)
    c1_ref[...] = c1
    c2_ref[...] = c2

    rows_e = 2 * CAP + 8
    ok1 = r1i < CAP
    ok2 = r2i < CAP
    row1 = i1 * rows_e + r1i
    row2 = i2 * rows_e + CAP + r2i
    d0_ref[...] = jnp.where(ok1, row1, i1 * rows_e + 2 * CAP).reshape(R // 128, 128)
    d1_ref[...] = jnp.where(ok2, row2, i2 * rows_e + 2 * CAP).reshape(R // 128, 128)
    s0_ref[...] = jnp.where(ok1, row1, 0).reshape(R // 128, 128)
    s1_ref[...] = jnp.where(ok2, row2, 0).reshape(R // 128, 128)
    wv_ref[...] = _pack2(jnp.where(ok1, v1, 0.0), jnp.where(ok2, v2, 0.0))
    probs_ref[...] = _pack2(v1, v2)
    idx_ref[...] = _pack2(i1, i2)

    @pl.when(b == NB - 1)
    def _ema():
        cnt = c1 + c2
        load = cnt / jnp.float32(2 * T)
        ema_out_ref[...] = _MOM * ema_ref[...] + (1.0 - _MOM) * load


def _dispatch_body(xf_hbm, d0_hbm, d1_hbm, xd_hbm, i0_v, i1_v, rows_v, sem,
                   *, CH, NC):
    wid = lax.axis_index("s") * NC + lax.axis_index("c")
    base = wid * CH
    row = base // 128
    off = base - row * 128
    l0 = pltpu.async_copy(d0_hbm.at[row, pl.ds(off, CH)], i0_v, sem)
    l1 = pltpu.async_copy(d1_hbm.at[row, pl.ds(off, CH)], i1_v, sem)
    l2 = pltpu.async_copy(xf_hbm.at[pl.ds(base, CH)], rows_v, sem)
    l0.wait()
    l1.wait()
    l2.wait()
    c0 = pltpu.async_copy(rows_v, xd_hbm.at[i0_v], sem)
    c1 = pltpu.async_copy(rows_v, xd_hbm.at[i1_v], sem)
    c0.wait()
    c1.wait()


def _gatherback_body(ty_hbm, s0_hbm, s1_hbm, g0_hbm, g1_hbm, i0_v, i1_v,
                     rows0_v, rows1_v, sem, *, CH, NC):
    wid = lax.axis_index("s") * NC + lax.axis_index("c")
    base = wid * CH
    row = base // 128
    off = base - row * 128
    l0 = pltpu.async_copy(s0_hbm.at[row, pl.ds(off, CH)], i0_v, sem)
    l1 = pltpu.async_copy(s1_hbm.at[row, pl.ds(off, CH)], i1_v, sem)
    l0.wait()
    l1.wait()
    c0 = pltpu.async_copy(ty_hbm.at[i0_v], rows0_v, sem)
    c1 = pltpu.async_copy(ty_hbm.at[i1_v], rows1_v, sem)
    c0.wait()
    c1.wait()
    w0 = pltpu.async_copy(rows0_v, g0_hbm.at[pl.ds(base, CH)], sem)
    w1 = pltpu.async_copy(rows1_v, g1_hbm.at[pl.ds(base, CH)], sem)
    w0.wait()
    w1.wait()


def _mlp_body(xd_ref, w1_ref, b1_ref, w2_ref, b2_ref, ty_ref):
    xb = xd_ref[0].astype(jnp.bfloat16)
    h = jnp.dot(xb, w1_ref[0].astype(jnp.bfloat16),
                preferred_element_type=jnp.float32) + b1_ref[0]
    h = 0.5 * h * (1.0 + lax.erf(h * jnp.float32(0.7071067811865476)))
    y = jnp.dot(h.astype(jnp.bfloat16), w2_ref[0].astype(jnp.bfloat16),
                preferred_element_type=jnp.float32) + b2_ref[0]
    ty_ref[0] = y


def _combine_body(g0_ref, g1_ref, wv_ref, out_ref):
    w0 = wv_ref[:, 0:1]
    w1 = wv_ref[:, 1:2]
    a = jnp.where(w0 > 0.0, w0 * g0_ref[...], 0.0)
    b = jnp.where(w1 > 0.0, w1 * g1_ref[...], 0.0)
    out_ref[...] = a + b


def kernel(x, Wg, W1, b1, W2, b2, ema_load):
    B, S, D = x.shape
    T = B * S
    E = Wg.shape[1]
    H = W1.shape[2]
    K = 2
    CAP = max(1, int(T * _CAP_F / E))
    ROWS_E = 2 * CAP + 8
    NPAD = E * ROWS_E
    R = 1024 if T % 1024 == 0 else T
    xf = x.reshape(T, D)

    NB = T // R
    router = pl.pallas_call(
        functools.partial(_router_body, T=T, E=E, CAP=CAP, R=R, NB=NB),
        grid=(NB,),
        in_specs=[
            pl.BlockSpec((R, D), lambda b: (b, 0)),
            pl.BlockSpec((D, E), lambda b: (0, 0)),
            pl.BlockSpec((1, E), lambda b: (0, 0)),
        ],
        out_specs=[
            pl.BlockSpec((R, 2), lambda b: (b, 0)),
            pl.BlockSpec((R, 2), lambda b: (b, 0)),
            pl.BlockSpec((R // 128, 128), lambda b: (b, 0)),
            pl.BlockSpec((R // 128, 128), lambda b: (b, 0)),
            pl.BlockSpec((R // 128, 128), lambda b: (b, 0)),
            pl.BlockSpec((R // 128, 128), lambda b: (b, 0)),
            pl.BlockSpec((R, 2), lambda b: (b, 0)),
            pl.BlockSpec((1, E), lambda b: (0, 0)),
        ],
        scratch_shapes=[
            pltpu.VMEM((1, E), jnp.float32),
            pltpu.VMEM((1, E), jnp.float32),
        ],
        out_shape=[
            jax.ShapeDtypeStruct((T, 2), jnp.float32),
            jax.ShapeDtypeStruct((T, 2), jnp.int32),
            jax.ShapeDtypeStruct((T // 128, 128), jnp.int32),
            jax.ShapeDtypeStruct((T // 128, 128), jnp.int32),
            jax.ShapeDtypeStruct((T // 128, 128), jnp.int32),
            jax.ShapeDtypeStruct((T // 128, 128), jnp.int32),
            jax.ShapeDtypeStruct((T, 2), jnp.float32),
            jax.ShapeDtypeStruct((1, E), jnp.float32),
        ],
    )
    (probs2, idx2, dst0, dst1, src0, src1, wv2,
     ema2) = router(xf, Wg, ema_load.reshape(1, E))

    info = plsc.get_sparse_core_info()
    NC, NS = info.num_cores, info.num_subcores
    NW = NC * NS
    CH = T // NW
    mesh = plsc.VectorSubcoreMesh(core_axis_name="c", subcore_axis_name="s")

    dispatch = pl.kernel(
        functools.partial(_dispatch_body, CH=CH, NC=NC),
        out_type=jax.ShapeDtypeStruct((NPAD, D), jnp.float32),
        mesh=mesh,
        scratch_types=[
            pltpu.VMEM((CH,), jnp.int32),
            pltpu.VMEM((CH,), jnp.int32),
            pltpu.VMEM((CH, D), jnp.float32),
            pltpu.SemaphoreType.DMA,
        ],
    )
    xd = dispatch(xf, dst0, dst1)

    mlp = pl.pallas_call(
        _mlp_body,
        grid=(E,),
        in_specs=[
            pl.BlockSpec((1, ROWS_E, D), lambda e: (e, 0, 0)),
            pl.BlockSpec((1, D, H), lambda e: (e, 0, 0)),
            pl.BlockSpec((1, 1, H), lambda e: (e, 0, 0)),
            pl.BlockSpec((1, H, D), lambda e: (e, 0, 0)),
            pl.BlockSpec((1, 1, D), lambda e: (e, 0, 0)),
        ],
        out_specs=pl.BlockSpec((1, ROWS_E, D), lambda e: (e, 0, 0)),
        out_shape=jax.ShapeDtypeStruct((E, ROWS_E, D), jnp.float32),
    )
    ty = mlp(xd.reshape(E, ROWS_E, D), W1, b1.reshape(E, 1, H),
             W2, b2.reshape(E, 1, D))

    gatherback = pl.kernel(
        functools.partial(_gatherback_body, CH=CH, NC=NC),
        out_type=[
            jax.ShapeDtypeStruct((T, D), jnp.float32),
            jax.ShapeDtypeStruct((T, D), jnp.float32),
        ],
        mesh=mesh,
        scratch_types=[
            pltpu.VMEM((CH,), jnp.int32),
            pltpu.VMEM((CH,), jnp.int32),
            pltpu.VMEM((CH, D), jnp.float32),
            pltpu.VMEM((CH, D), jnp.float32),
            pltpu.SemaphoreType.DMA,
        ],
    )
    g0, g1 = gatherback(ty.reshape(NPAD, D), src0, src1)

    RB = 256 if T % 256 == 0 else T
    combine = pl.pallas_call(
        _combine_body,
        grid=(T // RB,),
        in_specs=[
            pl.BlockSpec((RB, D), lambda i: (i, 0)),
            pl.BlockSpec((RB, D), lambda i: (i, 0)),
            pl.BlockSpec((RB, 2), lambda i: (i, 0)),
        ],
        out_specs=pl.BlockSpec((RB, D), lambda i: (i, 0)),
        out_shape=jax.ShapeDtypeStruct((T, D), jnp.float32),
    )
    out = combine(g0, g1, wv2)

    return (out.reshape(B, S, D), probs2.reshape(B, S, K),
            idx2.reshape(B, S, K), ema2.reshape(E))


# speedup vs baseline: 1.1246x; 1.0228x over previous
import functools

import jax
import jax.numpy as jnp
from jax import lax
from jax.experimental import pallas as pl
from jax.experimental.pallas import tpu as pltpu
from jax.experimental.pallas import tpu_sc as plsc

_MOM = 0.95
_CAP_F = 1.25


def _pack2(a, b):
    col = lax.broadcasted_iota(jnp.int32, (a.shape[0], 2), 1)
    return jnp.where(col == 0, a, b)


def _router_body(x_ref, wg_ref, ema_ref, probs_ref, idx_ref, d0_ref, d1_ref,
                 s0_ref, s1_ref, wv_ref, ema_out_ref, c1_ref, c2_ref,
                 *, T, E, CAP, R, NB):
    b = pl.program_id(0)

    @pl.when(b == 0)
    def _init():
        c1_ref[...] = jnp.zeros((1, E), jnp.float32)
        c2_ref[...] = jnp.zeros((1, E), jnp.float32)

    xb = x_ref[...]
    wg = wg_ref[...]
    logits = jnp.dot(xb, wg, preferred_element_type=jnp.float32)
    m = jnp.max(logits, axis=1, keepdims=True)
    p = jnp.exp(logits - m)
    probs_full = p / jnp.sum(p, axis=1, keepdims=True)

    ei = lax.broadcasted_iota(jnp.int32, (R, E), 1)
    v1 = jnp.max(probs_full, axis=1, keepdims=True)
    i1 = jnp.min(jnp.where(probs_full == v1, ei, E), axis=1, keepdims=True)
    masked = jnp.where(ei == i1, jnp.float32(-1.0), probs_full)
    v2 = jnp.max(masked, axis=1, keepdims=True)
    i2 = jnp.min(jnp.where(masked == v2, ei, E), axis=1, keepdims=True)

    oh1 = (ei == i1).astype(jnp.float32)
    oh2 = (ei == i2).astype(jnp.float32)

    ri = lax.broadcasted_iota(jnp.int32, (R, R), 0)
    ci = lax.broadcasted_iota(jnp.int32, (R, R), 1)
    strict = (ci < ri).astype(jnp.float32)
    c1 = c1_ref[...]
    c2 = c2_ref[...]
    rb1 = jnp.dot(strict, oh1, preferred_element_type=jnp.float32) + c1
    rb2 = jnp.dot(strict, oh2, preferred_element_type=jnp.float32) + c2
    r1i = jnp.sum(oh1 * rb1, axis=1, keepdims=True).astype(jnp.int32)
    r2i = jnp.sum(oh2 * rb2, axis=1, keepdims=True).astype(jnp.int32)
    c1 = c1 + jnp.sum(oh1, axis=0, keepdims=True)
    c2 = c2 + jnp.sum(oh2, axis=0, keepdims=True)
    c1_ref[...] = c1
    c2_ref[...] = c2

    rows_e = 2 * CAP + 8
    ok1 = r1i < CAP
    ok2 = r2i < CAP
    row1 = i1 * rows_e + r1i
    row2 = i2 * rows_e + CAP + r2i
    d0_ref[...] = jnp.where(ok1, row1, i1 * rows_e + 2 * CAP).reshape(R // 128, 128)
    d1_ref[...] = jnp.where(ok2, row2, i2 * rows_e + 2 * CAP).reshape(R // 128, 128)
    s0_ref[...] = jnp.where(ok1, row1, 0).reshape(R // 128, 128)
    s1_ref[...] = jnp.where(ok2, row2, 0).reshape(R // 128, 128)
    wv_ref[...] = _pack2(jnp.where(ok1, v1, 0.0), jnp.where(ok2, v2, 0.0))
    probs_ref[...] = _pack2(v1, v2)
    idx_ref[...] = _pack2(i1, i2)

    @pl.when(b == NB - 1)
    def _ema():
        cnt = c1 + c2
        load = cnt / jnp.float32(2 * T)
        ema_out_ref[...] = _MOM * ema_ref[...] + (1.0 - _MOM) * load


def _dispatch_body(xf_hbm, d0_hbm, d1_hbm, xd_hbm, i0_v, i1_v, rows_v, sem,
                   *, CH, NC):
    wid = lax.axis_index("s") * NC + lax.axis_index("c")
    base = wid * CH
    row = base // 128
    off = base - row * 128
    l0 = pltpu.async_copy(d0_hbm.at[row, pl.ds(off, CH)], i0_v, sem)
    l1 = pltpu.async_copy(d1_hbm.at[row, pl.ds(off, CH)], i1_v, sem)
    l2 = pltpu.async_copy(xf_hbm.at[pl.ds(base, CH)], rows_v, sem)
    l0.wait()
    l1.wait()
    l2.wait()
    c0 = pltpu.async_copy(rows_v, xd_hbm.at[i0_v], sem)
    c1 = pltpu.async_copy(rows_v, xd_hbm.at[i1_v], sem)
    c0.wait()
    c1.wait()


def _gatherback_body(ty_hbm, s0_hbm, s1_hbm, g0_hbm, g1_hbm, i0_v, i1_v,
                     rows0_v, rows1_v, sem, *, CH, NC):
    wid = lax.axis_index("s") * NC + lax.axis_index("c")
    base = wid * CH
    row = base // 128
    off = base - row * 128
    l0 = pltpu.async_copy(s0_hbm.at[row, pl.ds(off, CH)], i0_v, sem)
    l1 = pltpu.async_copy(s1_hbm.at[row, pl.ds(off, CH)], i1_v, sem)
    l0.wait()
    l1.wait()
    c0 = pltpu.async_copy(ty_hbm.at[i0_v], rows0_v, sem)
    c1 = pltpu.async_copy(ty_hbm.at[i1_v], rows1_v, sem)
    c0.wait()
    c1.wait()
    w0 = pltpu.async_copy(rows0_v, g0_hbm.at[pl.ds(base, CH)], sem)
    w1 = pltpu.async_copy(rows1_v, g1_hbm.at[pl.ds(base, CH)], sem)
    w0.wait()
    w1.wait()


def _mlp_body(xd_ref, w1_ref, b1_ref, w2_ref, b2_ref, ty_ref):
    xb = xd_ref[0].astype(jnp.bfloat16)
    h = jnp.dot(xb, w1_ref[0].astype(jnp.bfloat16),
                preferred_element_type=jnp.float32) + b1_ref[0]
    h = 0.5 * h * (1.0 + lax.erf(h * jnp.float32(0.7071067811865476)))
    y = jnp.dot(h.astype(jnp.bfloat16), w2_ref[0].astype(jnp.bfloat16),
                preferred_element_type=jnp.float32) + b2_ref[0]
    ty_ref[0] = y


def _combine_body(g0_ref, g1_ref, wv_ref, out_ref):
    w0 = wv_ref[:, 0:1]
    w1 = wv_ref[:, 1:2]
    a = jnp.where(w0 > 0.0, w0 * g0_ref[...], 0.0)
    b = jnp.where(w1 > 0.0, w1 * g1_ref[...], 0.0)
    out_ref[...] = a + b


def kernel(x, Wg, W1, b1, W2, b2, ema_load):
    B, S, D = x.shape
    T = B * S
    E = Wg.shape[1]
    H = W1.shape[2]
    K = 2
    CAP = max(1, int(T * _CAP_F / E))
    ROWS_E = 2 * CAP + 8
    NPAD = E * ROWS_E
    R = 1024 if T % 1024 == 0 else T
    xf = x.reshape(T, D)

    NB = T // R
    router = pl.pallas_call(
        functools.partial(_router_body, T=T, E=E, CAP=CAP, R=R, NB=NB),
        grid=(NB,),
        in_specs=[
            pl.BlockSpec((R, D), lambda b: (b, 0)),
            pl.BlockSpec((D, E), lambda b: (0, 0)),
            pl.BlockSpec((1, E), lambda b: (0, 0)),
        ],
        out_specs=[
            pl.BlockSpec((R, 2), lambda b: (b, 0)),
            pl.BlockSpec((R, 2), lambda b: (b, 0)),
            pl.BlockSpec((R // 128, 128), lambda b: (b, 0)),
            pl.BlockSpec((R // 128, 128), lambda b: (b, 0)),
            pl.BlockSpec((R // 128, 128), lambda b: (b, 0)),
            pl.BlockSpec((R // 128, 128), lambda b: (b, 0)),
            pl.BlockSpec((R, 2), lambda b: (b, 0)),
            pl.BlockSpec((1, E), lambda b: (0, 0)),
        ],
        scratch_shapes=[
            pltpu.VMEM((1, E), jnp.float32),
            pltpu.VMEM((1, E), jnp.float32),
        ],
        out_shape=[
            jax.ShapeDtypeStruct((T, 2), jnp.float32),
            jax.ShapeDtypeStruct((T, 2), jnp.int32),
            jax.ShapeDtypeStruct((T // 128, 128), jnp.int32),
            jax.ShapeDtypeStruct((T // 128, 128), jnp.int32),
            jax.ShapeDtypeStruct((T // 128, 128), jnp.int32),
            jax.ShapeDtypeStruct((T // 128, 128), jnp.int32),
            jax.ShapeDtypeStruct((T, 2), jnp.float32),
            jax.ShapeDtypeStruct((1, E), jnp.float32),
        ],
    )
    (probs2, idx2, dst0, dst1, src0, src1, wv2,
     ema2) = router(xf, Wg, ema_load.reshape(1, E))

    info = plsc.get_sparse_core_info()
    NC, NS = info.num_cores, info.num_subcores
    NW = NC * NS
    CH = T // NW
    mesh = plsc.VectorSubcoreMesh(core_axis_name="c", subcore_axis_name="s")

    dispatch = pl.kernel(
        functools.partial(_dispatch_body, CH=CH, NC=NC),
        out_type=jax.ShapeDtypeStruct((NPAD, D), jnp.float32),
        mesh=mesh,
        scratch_types=[
            pltpu.VMEM((CH,), jnp.int32),
            pltpu.VMEM((CH,), jnp.int32),
            pltpu.VMEM((CH, D), jnp.float32),
            pltpu.SemaphoreType.DMA,
        ],
    )
    xd = dispatch(xf, dst0, dst1)

    mlp = pl.pallas_call(
        _mlp_body,
        grid=(E,),
        in_specs=[
            pl.BlockSpec((1, ROWS_E, D), lambda e: (e, 0, 0)),
            pl.BlockSpec((1, D, H), lambda e: (e, 0, 0)),
            pl.BlockSpec((1, 1, H), lambda e: (e, 0, 0)),
            pl.BlockSpec((1, H, D), lambda e: (e, 0, 0)),
            pl.BlockSpec((1, 1, D), lambda e: (e, 0, 0)),
        ],
        out_specs=pl.BlockSpec((1, ROWS_E, D), lambda e: (e, 0, 0)),
        out_shape=jax.ShapeDtypeStruct((E, ROWS_E, D), jnp.float32),
    )
    ty = mlp(xd.reshape(E, ROWS_E, D), W1, b1.reshape(E, 1, H),
             W2, b2.reshape(E, 1, D))

    gatherback = pl.kernel(
        functools.partial(_gatherback_body, CH=CH, NC=NC),
        out_type=[
            jax.ShapeDtypeStruct((T, D), jnp.float32),
            jax.ShapeDtypeStruct((T, D), jnp.float32),
        ],
        mesh=mesh,
        scratch_types=[
            pltpu.VMEM((CH,), jnp.int32),
            pltpu.VMEM((CH,), jnp.int32),
            pltpu.VMEM((CH, D), jnp.float32),
            pltpu.VMEM((CH, D), jnp.float32),
            pltpu.SemaphoreType.DMA,
        ],
    )
    g0, g1 = gatherback(ty.reshape(NPAD, D), src0, src1)

    RB = 1024 if T % 1024 == 0 else T
    combine = pl.pallas_call(
        _combine_body,
        grid=(T // RB,),
        in_specs=[
            pl.BlockSpec((RB, D), lambda i: (i, 0)),
            pl.BlockSpec((RB, D), lambda i: (i, 0)),
            pl.BlockSpec((RB, 2), lambda i: (i, 0)),
        ],
        out_specs=pl.BlockSpec((RB, D), lambda i: (i, 0)),
        out_shape=jax.ShapeDtypeStruct((T, D), jnp.float32),
    )
    out = combine(g0, g1, wv2)

    return (out.reshape(B, S, D), probs2.reshape(B, S, K),
            idx2.reshape(B, S, K), ema2.reshape(E))
